# baseline (speedup 1.0000x reference)
import jax
import jax.numpy as jnp
from jax.experimental import pallas as pl
from jax.experimental.pallas import tpu as pltpu

_B, _N, _D = 2, 1024, 768
_K = 512
_C = 151
_EPS = 1e-5
_F32 = jnp.float32


def _gelu(x):
    return 0.5 * x * (1.0 + jax.lax.erf(x * (2.0 ** -0.5)))


def _ln(x, g=None, b=None):
    m = jnp.mean(x, axis=-1, keepdims=True)
    xc = x - m
    v = jnp.mean(xc * xc, axis=-1, keepdims=True)
    y = xc * jax.lax.rsqrt(v + _EPS)
    if g is not None:
        y = y * g + b
    return y


def _mm_t(a, w):
    return jax.lax.dot_general(a, w, (((1,), (1,)), ((), ())),
                               preferred_element_type=_F32)


def _mm_tb(a, w):
    return jax.lax.dot_general(a.astype(jnp.bfloat16), w.astype(jnp.bfloat16),
                               (((1,), (1,)), ((), ())),
                               preferred_element_type=_F32)


def _fused_kernel(x_ref, sm_ref,
                  sw1, sw2, sw3, ow1, ow2, ow3, w1, w2, cw,
                  probs_ref, bbox_ref, wbuf, sems):
    x = x_ref[0]

    w_hbm = (sw1, sw2, sw3, ow1, ow2, ow3, w1, w2)
    for i, wr in enumerate(w_hbm):
        pltpu.make_async_copy(wr, wbuf.at[i], sems.at[i]).start()

    def wmat(i):
        pltpu.make_async_copy(w_hbm[i], wbuf.at[i], sems.at[i]).wait()
        return wbuf[i]

    def vrow(i):
        return sm_ref[i:i + 1, :]

    def head(i0, ib):
        h = _gelu(_mm_tb(x, wmat(i0)) + vrow(ib))
        h = _gelu(_mm_tb(h, wmat(i0 + 1)) + vrow(ib + 1))
        h = _gelu(_mm_tb(h, wmat(i0 + 2)) + vrow(ib + 2))
        return _ln(h, vrow(ib + 3), vrow(ib + 4))

    q = x + head(0, 0)
    k = x + head(3, 5)

    st = jax.lax.dot_general(k.astype(jnp.bfloat16), q.astype(jnp.bfloat16),
                             (((1,), (1,)), ((), ())),
                             preferred_element_type=_F32)
    ii = jax.lax.broadcasted_iota(jnp.int32, (_N, _N), 0)
    jj = jax.lax.broadcasted_iota(jnp.int32, (_N, _N), 1)
    m = jnp.max(st, axis=0, keepdims=True)
    z = jnp.sum(jnp.exp(st - m), axis=0, keepdims=True)
    sd = jnp.sum(jnp.where(ii == jj, st, 0.0), axis=0, keepdims=True)
    d = jnp.exp(sd - m) / z

    diag_d = jnp.where(ii == jj, d, 0.0)
    dcol = jnp.dot(diag_d, jnp.ones((_N, 128), _F32),
                   preferred_element_type=_F32)
    di = pltpu.repeat(dcol, 8, axis=1)
    beats1 = jnp.where((d > di) | ((d == di) & (jj < ii)), 1.0, 0.0)
    rank_col = jnp.dot(beats1, jnp.ones((_N, 128), _F32),
                       preferred_element_type=_F32)
    sel_col = jnp.where(rank_col < float(_K), 1.0, 0.0)
    sel_coln = pltpu.repeat(sel_col, 8, axis=1)
    beats2 = jnp.where(ii == jj, 0.0, 1.0 - beats1)
    rank_row = jnp.sum(beats2, axis=0, keepdims=True)
    sel_row = jnp.where(rank_row < float(_K), 1.0, 0.0)
    psel = jnp.sum(jnp.where(ii <= jj, sel_coln, 0.0),
                   axis=0, keepdims=True)

    ss = jax.lax.broadcasted_iota(jnp.int32, (_K, _N), 0).astype(_F32)
    oh = jnp.where((sel_row > 0.0) & (psel == ss + 1.0), 1.0, 0.0)
    q_sel = jnp.dot(oh, q, preferred_element_type=_F32)

    h = _ln(q_sel + q_sel)
    h = _gelu(_mm_tb(h, wmat(6)) + vrow(10))
    h = _mm_tb(h, wmat(7)) + vrow(11)
    o = _ln(h, vrow(12), vrow(13))

    bbox_w = sm_ref[16:20, :]
    bbox_b = sm_ref[15:16, :4]
    bbox_ref[0] = jax.nn.relu(_mm_t(o, bbox_w) + bbox_b)
    lg = _mm_t(o, cw[...]) + sm_ref[14:15, :_C]
    lm = jnp.max(lg, axis=-1, keepdims=True)
    e = jnp.exp(lg - lm)
    probs_ref[0] = e / jnp.sum(e, axis=-1, keepdims=True)


def _full_spec(shape):
    return pl.BlockSpec(shape, lambda *_: (0,) * len(shape))


def kernel(x, params):
    sh, oh_, m2 = params['subject_head'], params['object_head'], params['mlp2']

    def pad_row(v):
        return jnp.pad(v, (0, _D - v.shape[0]))

    smalls = jnp.concatenate([
        jnp.stack([sh['b1'], sh['b2'], sh['b3'], sh['g'], sh['be'],
                   oh_['b1'], oh_['b2'], oh_['b3'], oh_['g'], oh_['be'],
                   m2['b1'], m2['b2'], m2['g'], m2['be'],
                   pad_row(params['cls_b']), pad_row(params['bbox_b'])]),
        params['bbox_w'],
    ], axis=0)

    hbm_spec = pl.BlockSpec(memory_space=pl.ANY)
    w_args = [sh['w1'], sh['w2'], sh['w3'], oh_['w1'], oh_['w2'], oh_['w3'],
              m2['w1'], m2['w2']]

    probs, bbox = pl.pallas_call(
        _fused_kernel,
        grid=(_B,),
        in_specs=[pl.BlockSpec((1, _N, _D), lambda b: (b, 0, 0)),
                  _full_spec((20, _D))] + [hbm_spec] * 8 + [
                  _full_spec((_C, _D))],
        out_specs=[pl.BlockSpec((1, _K, _C), lambda b: (b, 0, 0)),
                   pl.BlockSpec((1, _K, 4), lambda b: (b, 0, 0))],
        out_shape=[jax.ShapeDtypeStruct((_B, _K, _C), _F32),
                   jax.ShapeDtypeStruct((_B, _K, 4), _F32)],
        scratch_shapes=[pltpu.VMEM((8, _D, _D), _F32),
                        pltpu.SemaphoreType.DMA((8,))],
        compiler_params=pltpu.CompilerParams(
            dimension_semantics=("parallel",),
            vmem_limit_bytes=58 * 1024 * 1024,
        ),
    )(x, smalls, *w_args, params['cls_w'])

    return probs, bbox


# speedup vs baseline: 1.1195x; 1.1195x over previous
import jax
import jax.numpy as jnp
from jax.experimental import pallas as pl
from jax.experimental.pallas import tpu as pltpu

_B, _N, _D = 2, 1024, 768
_K = 512
_C = 151
_EPS = 1e-5
_F32 = jnp.float32


def _gelu(x):
    return 0.5 * x * (1.0 + jax.lax.erf(x * (2.0 ** -0.5)))


def _ln(x, g=None, b=None):
    m = jnp.mean(x, axis=-1, keepdims=True)
    xc = x - m
    v = jnp.mean(xc * xc, axis=-1, keepdims=True)
    y = xc * jax.lax.rsqrt(v + _EPS)
    if g is not None:
        y = y * g + b
    return y


def _mm_t(a, w):
    return jax.lax.dot_general(a, w, (((1,), (1,)), ((), ())),
                               preferred_element_type=_F32)


def _mm_tb(a, w):
    return jax.lax.dot_general(a.astype(jnp.bfloat16), w.astype(jnp.bfloat16),
                               (((1,), (1,)), ((), ())),
                               preferred_element_type=_F32)


def _fused_kernel(x_ref, sm_ref,
                  sw1, sw2, sw3, ow1, ow2, ow3, w1, w2, cw,
                  probs_ref, bbox_ref):
    x = x_ref[0]

    def vrow(i):
        return sm_ref[i:i + 1, :]

    def head(hw1, hw2, hw3, ib):
        h = _gelu(_mm_tb(x, hw1[...]) + vrow(ib))
        h = _gelu(_mm_tb(h, hw2[...]) + vrow(ib + 1))
        h = _gelu(_mm_tb(h, hw3[...]) + vrow(ib + 2))
        return _ln(h, vrow(ib + 3), vrow(ib + 4))

    q = x + head(sw1, sw2, sw3, 0)
    k = x + head(ow1, ow2, ow3, 5)

    st = jax.lax.dot_general(k.astype(jnp.bfloat16), q.astype(jnp.bfloat16),
                             (((1,), (1,)), ((), ())),
                             preferred_element_type=_F32)
    ii = jax.lax.broadcasted_iota(jnp.int32, (_N, _N), 0)
    jj = jax.lax.broadcasted_iota(jnp.int32, (_N, _N), 1)
    m = jnp.max(st, axis=0, keepdims=True)
    z = jnp.sum(jnp.exp(st - m), axis=0, keepdims=True)
    sd = jnp.sum(jnp.where(ii == jj, st, 0.0), axis=0, keepdims=True)
    d = jnp.exp(sd - m) / z

    diag_d = jnp.where(ii == jj, d, 0.0)
    dcol = jnp.dot(diag_d, jnp.ones((_N, 128), _F32),
                   preferred_element_type=_F32)
    di = pltpu.repeat(dcol, 8, axis=1)
    beats1 = jnp.where((d > di) | ((d == di) & (jj < ii)), 1.0, 0.0)
    rank_col = jnp.dot(beats1, jnp.ones((_N, 128), _F32),
                       preferred_element_type=_F32)
    sel_col = jnp.where(rank_col < float(_K), 1.0, 0.0)
    sel_coln = pltpu.repeat(sel_col, 8, axis=1)
    beats2 = jnp.where(ii == jj, 0.0, 1.0 - beats1)
    rank_row = jnp.sum(beats2, axis=0, keepdims=True)
    sel_row = jnp.where(rank_row < float(_K), 1.0, 0.0)
    psel = jnp.sum(jnp.where(ii <= jj, sel_coln, 0.0),
                   axis=0, keepdims=True)

    ss = jax.lax.broadcasted_iota(jnp.int32, (_K, _N), 0).astype(_F32)
    oh = jnp.where((sel_row > 0.0) & (psel == ss + 1.0), 1.0, 0.0)
    q_sel = jnp.dot(oh, q, preferred_element_type=_F32)

    h = _ln(q_sel + q_sel)
    h = _gelu(_mm_tb(h, w1[...]) + vrow(10))
    h = _mm_tb(h, w2[...]) + vrow(11)
    o = _ln(h, vrow(12), vrow(13))

    bbox_w = sm_ref[16:20, :]
    bbox_b = sm_ref[15:16, :4]
    bbox_ref[0] = jax.nn.relu(_mm_t(o, bbox_w) + bbox_b)
    lg = _mm_t(o, cw[...]) + sm_ref[14:15, :_C]
    lm = jnp.max(lg, axis=-1, keepdims=True)
    e = jnp.exp(lg - lm)
    probs_ref[0] = e / jnp.sum(e, axis=-1, keepdims=True)


def _full_spec(shape):
    return pl.BlockSpec(shape, lambda *_: (0,) * len(shape))


def kernel(x, params):
    sh, oh_, m2 = params['subject_head'], params['object_head'], params['mlp2']

    def pad_row(v):
        return jnp.pad(v, (0, _D - v.shape[0]))

    smalls = jnp.concatenate([
        jnp.stack([sh['b1'], sh['b2'], sh['b3'], sh['g'], sh['be'],
                   oh_['b1'], oh_['b2'], oh_['b3'], oh_['g'], oh_['be'],
                   m2['b1'], m2['b2'], m2['g'], m2['be'],
                   pad_row(params['cls_b']), pad_row(params['bbox_b'])]),
        params['bbox_w'],
    ], axis=0)

    w_args = [sh['w1'], sh['w2'], sh['w3'], oh_['w1'], oh_['w2'], oh_['w3'],
              m2['w1'], m2['w2']]

    probs, bbox = pl.pallas_call(
        _fused_kernel,
        grid=(_B,),
        in_specs=[pl.BlockSpec((1, _N, _D), lambda b: (b, 0, 0)),
                  _full_spec((20, _D))] +
                 [_full_spec((_D, _D))] * 8 +
                 [_full_spec((_C, _D))],
        out_specs=[pl.BlockSpec((1, _K, _C), lambda b: (b, 0, 0)),
                   pl.BlockSpec((1, _K, 4), lambda b: (b, 0, 0))],
        out_shape=[jax.ShapeDtypeStruct((_B, _K, _C), _F32),
                   jax.ShapeDtypeStruct((_B, _K, 4), _F32)],
        compiler_params=pltpu.CompilerParams(
            dimension_semantics=("parallel",),
            vmem_limit_bytes=58 * 1024 * 1024,
        ),
    )(x, smalls, *w_args, params['cls_w'])

    return probs, bbox
